# baseline (speedup 1.0000x reference)
import functools

import jax
import jax.numpy as jnp
from jax import lax
from jax.experimental import pallas as pl
from jax.experimental.pallas import tpu as pltpu
from jax.experimental.pallas import tpu_sc as plsc

N = 10000
E = 320000
D = 128

NC = 2
NS = 16
D2 = D // NC
ET = E // NS
CH = 128
EP = 20224
NCHUNK = EP // CH
NB = 5
NTAIL = NCHUNK - (NCHUNK // NB) * NB

NPAD = 10240
SEG = NPAD // NS

ROWS_T = N // NS

E_T = E // (NC * NS)
K1_CH = 2000
K1_NCHUNK = E_T // K1_CH

_mesh = plsc.VectorSubcoreMesh(core_axis_name="c", subcore_axis_name="s")
_sc_params = pltpu.CompilerParams(
    needs_layout_passes=False, use_tc_tiling_on_sc=False
)


@functools.partial(
    pl.kernel,
    mesh=_mesh,
    out_type=jax.ShapeDtypeStruct((NC, NPAD), jnp.float32),
    scratch_types=[
        pltpu.VMEM((K1_CH,), jnp.int32),
        pltpu.VMEM((NPAD,), jnp.float32),
        pltpu.VMEM((SEG,), jnp.float32),
        pltpu.VMEM((SEG,), jnp.float32),
        pltpu.VMEM_SHARED((NS, NPAD), jnp.float32),
    ],
    compiler_params=_sc_params,
)
def _deg_kernel(dst_hbm, deg_hbm, idx_v, hist_v, seg_v, acc_v, stage_s):
    cid = lax.axis_index("c")
    sid = lax.axis_index("s")
    ones = jnp.ones((16,), jnp.float32)

    def zero_body(i, _):
        hist_v[pl.ds(i * 16, 16)] = jnp.zeros((16,), jnp.float32)
        return ()
    lax.fori_loop(0, NPAD // 16, zero_body, ())

    def chunk_body(j, _):
        base = (cid * NS + sid) * E_T + j * K1_CH
        pltpu.sync_copy(dst_hbm.at[pl.ds(base, K1_CH)], idx_v)

        def vec_body(k, _):
            idx = idx_v[pl.ds(k * 16, 16)]
            plsc.addupdate_scatter(hist_v, [idx], ones)
            return ()
        lax.fori_loop(0, K1_CH // 16, vec_body, ())
        return ()
    lax.fori_loop(0, K1_NCHUNK, chunk_body, ())

    pltpu.sync_copy(hist_v, stage_s.at[sid])

    plsc.subcore_barrier()

    def rzero(i, _):
        acc_v[pl.ds(i * 16, 16)] = jnp.zeros((16,), jnp.float32)
        return ()
    lax.fori_loop(0, SEG // 16, rzero, ())

    def radd(j, _):
        pltpu.sync_copy(stage_s.at[j, pl.ds(sid * SEG, SEG)], seg_v)

        def vadd(k, _):
            acc_v[pl.ds(k * 16, 16)] = acc_v[pl.ds(k * 16, 16)] + seg_v[pl.ds(k * 16, 16)]
            return ()
        lax.fori_loop(0, SEG // 16, vadd, ())
        return ()
    lax.fori_loop(0, NS, radd, ())

    pltpu.sync_copy(acc_v, deg_hbm.at[cid, pl.ds(sid * SEG, SEG)])


@functools.partial(
    pl.kernel,
    mesh=_mesh,
    out_type=jax.ShapeDtypeStruct((N, D), jnp.float32),
    scratch_types=[
        pltpu.VMEM((NCHUNK, CH), jnp.int32),
        pltpu.VMEM((NCHUNK, CH), jnp.int32),
        pltpu.VMEM((NB, CH, D2), jnp.float32),
        pltpu.VMEM((ROWS_T + 16,), jnp.float32),
        pltpu.VMEM((D2,), jnp.float32),
        pltpu.VMEM_SHARED((N + 8, D2), jnp.float32),
        pltpu.SemaphoreType.DMA((NB,)),
        pltpu.SemaphoreType.DMA((NB,)),
    ],
    compiler_params=_sc_params,
)
def _msg_kernel(g2_hbm, src_hbm, dst_hbm, dinv_hbm, b_hbm, out_hbm,
                src_v, dst_v, rows_v, dinv_v, b_v, acc_s, gsem, ssem):
    cid = lax.axis_index("c")
    sid = lax.axis_index("s")

    pltpu.sync_copy(src_hbm.at[cid, sid], src_v)
    pltpu.sync_copy(dst_hbm.at[sid], dst_v)

    pltpu.sync_copy(dinv_hbm.at[sid], dinv_v.at[pl.ds(0, ROWS_T)])
    pltpu.sync_copy(b_hbm.at[cid], b_v)

    base = sid * ROWS_T
    pltpu.sync_copy(
        g2_hbm.at[pl.ds(cid * N + base, ROWS_T)], acc_s.at[pl.ds(base, ROWS_T)]
    )

    plsc.subcore_barrier()

    for b in range(NB):
        pltpu.async_copy(g2_hbm.at[src_v.at[b]], rows_v.at[b], gsem.at[b])

    def group_body(gi, _):
        for b in range(NB):
            c = gi * NB + b
            pltpu.make_async_copy(
                g2_hbm.at[src_v.at[c]], rows_v.at[b], gsem.at[b]
            ).wait()
            pltpu.async_copy(
                rows_v.at[b], acc_s.at[dst_v.at[c]], ssem.at[b], add=True
            )
        for b in range(NB):
            c = gi * NB + b
            pltpu.make_async_copy(
                rows_v.at[b], acc_s.at[dst_v.at[c]], ssem.at[b]
            ).wait()

            @pl.when(c + NB < NCHUNK)
            def _():
                pltpu.async_copy(
                    g2_hbm.at[src_v.at[c + NB]], rows_v.at[b], gsem.at[b]
                )
        return ()
    lax.fori_loop(0, NCHUNK // NB, group_body, ())

    for t in range(NTAIL):
        c = (NCHUNK // NB) * NB + t
        pltpu.make_async_copy(
            g2_hbm.at[src_v.at[c]], rows_v.at[t], gsem.at[t]
        ).wait()
        pltpu.async_copy(
            rows_v.at[t], acc_s.at[dst_v.at[c]], ssem.at[t], add=True
        )
    for t in range(NTAIL):
        c = (NCHUNK // NB) * NB + t
        pltpu.make_async_copy(
            rows_v.at[t], acc_s.at[dst_v.at[c]], ssem.at[t]
        ).wait()

    plsc.subcore_barrier()

    ebuf = rows_v.at[0]
    ecuts = [(0, CH), (CH, CH), (2 * CH, CH), (3 * CH, CH), (4 * CH, ROWS_T - 4 * CH)]
    for off, nr in ecuts:
        pltpu.sync_copy(acc_s.at[pl.ds(base + off, nr)], ebuf.at[pl.ds(0, nr)])

        def row_body(r, _):
            dv = dinv_v[pl.ds(off + r, 16)][0]
            for q in range(D2 // 16):
                v = ebuf[r, pl.ds(q * 16, 16)]
                s = v * dv + b_v[pl.ds(q * 16, 16)]
                ebuf[r, pl.ds(q * 16, 16)] = jnp.where(s >= 0.0, s, 0.2 * s)
            return ()
        lax.fori_loop(0, nr, row_body, ())

        pltpu.sync_copy(
            ebuf.at[pl.ds(0, nr)],
            out_hbm.at[pl.ds(base + off, nr), pl.ds(cid * D2, D2)],
        )


def _transform_body(x_ref, w_ref, wc_ref, deg0_ref, deg1_ref, g2_ref, dinv_ref):
    x0 = jnp.dot(x_ref[...], w_ref[...], preferred_element_type=jnp.float32)
    h = jnp.dot(x0, wc_ref[0], preferred_element_type=jnp.float32)
    dinv = lax.rsqrt(deg0_ref[...] + deg1_ref[...] + 1.0)
    g2_ref[...] = (h * dinv)[None]
    dinv_ref[...] = dinv


_BLK = 1000


def kernel(x, edge_index, W, Wc, b):
    src = edge_index[0]
    dst_flat = edge_index[1]
    src_both = jnp.concatenate(
        [
            jnp.stack([src, src + N]).reshape(NC, NS, ET),
            jnp.zeros((NC, NS, EP - ET), jnp.int32),
        ],
        axis=2,
    ).reshape(NC, NS, NCHUNK, CH)
    dst = jnp.concatenate(
        [
            dst_flat.reshape(NS, ET),
            jnp.full((NS, EP - ET), N, jnp.int32),
        ],
        axis=1,
    ).reshape(NS, NCHUNK, CH)

    deg2 = _deg_kernel(dst_flat)
    deg0 = deg2[0].reshape(NPAD, 1)
    deg1 = deg2[1].reshape(NPAD, 1)

    g2, dinv = pl.pallas_call(
        _transform_body,
        grid=(NC, N // _BLK),
        in_specs=[
            pl.BlockSpec((_BLK, D), lambda j, i: (i, 0)),
            pl.BlockSpec((D, D), lambda j, i: (0, 0)),
            pl.BlockSpec((1, D, D2), lambda j, i: (j, 0, 0)),
            pl.BlockSpec((_BLK, 1), lambda j, i: (i, 0)),
            pl.BlockSpec((_BLK, 1), lambda j, i: (i, 0)),
        ],
        out_specs=(
            pl.BlockSpec((1, _BLK, D2), lambda j, i: (j, i, 0)),
            pl.BlockSpec((_BLK, 1), lambda j, i: (i, 0)),
        ),
        out_shape=(
            jax.ShapeDtypeStruct((NC, N, D2), jnp.float32),
            jax.ShapeDtypeStruct((N, 1), jnp.float32),
        ),
    )(x, W, Wc.reshape(D, NC, D2).transpose(1, 0, 2), deg0, deg1)

    out = _msg_kernel(
        g2.reshape(NC * N, D2),
        src_both,
        dst,
        dinv.reshape(NS, ROWS_T),
        b.reshape(NC, D2),
    )
    return out


# speedup vs baseline: 1.5395x; 1.5395x over previous
import functools

import jax
import jax.numpy as jnp
from jax import lax
from jax.experimental import pallas as pl
from jax.experimental.pallas import tpu as pltpu
from jax.experimental.pallas import tpu_sc as plsc

N = 10000
E = 320000
D = 128

NC = 2
NS = 16
D2 = D // NC
ET = E // NS
CH = 80
NCHUNK = ET // CH
NB = 9
NTAIL = NCHUNK - (NCHUNK // NB) * NB

NPAD = 10240
SEG = NPAD // NS

ROWS_T = N // NS

E_T = E // (NC * NS)
K1_CH = 2000
K1_NCHUNK = E_T // K1_CH

_mesh = plsc.VectorSubcoreMesh(core_axis_name="c", subcore_axis_name="s")
_sc_params = pltpu.CompilerParams(
    needs_layout_passes=False, use_tc_tiling_on_sc=False
)


@functools.partial(
    pl.kernel,
    mesh=_mesh,
    out_type=jax.ShapeDtypeStruct((NC, NPAD), jnp.float32),
    scratch_types=[
        pltpu.VMEM((K1_CH,), jnp.int32),
        pltpu.VMEM((NPAD,), jnp.float32),
        pltpu.VMEM((SEG,), jnp.float32),
        pltpu.VMEM((SEG,), jnp.float32),
        pltpu.VMEM_SHARED((NS, NPAD), jnp.float32),
    ],
    compiler_params=_sc_params,
)
def _deg_kernel(dst_hbm, deg_hbm, idx_v, hist_v, seg_v, acc_v, stage_s):
    cid = lax.axis_index("c")
    sid = lax.axis_index("s")
    ones = jnp.ones((16,), jnp.float32)

    def zero_body(i, _):
        hist_v[pl.ds(i * 16, 16)] = jnp.zeros((16,), jnp.float32)
        return ()
    lax.fori_loop(0, NPAD // 16, zero_body, ())

    def chunk_body(j, _):
        base = (cid * NS + sid) * E_T + j * K1_CH
        pltpu.sync_copy(dst_hbm.at[pl.ds(base, K1_CH)], idx_v)

        def vec_body(k, _):
            idx = idx_v[pl.ds(k * 16, 16)]
            plsc.addupdate_scatter(hist_v, [idx], ones)
            return ()
        lax.fori_loop(0, K1_CH // 16, vec_body, ())
        return ()
    lax.fori_loop(0, K1_NCHUNK, chunk_body, ())

    pltpu.sync_copy(hist_v, stage_s.at[sid])

    plsc.subcore_barrier()

    def rzero(i, _):
        acc_v[pl.ds(i * 16, 16)] = jnp.zeros((16,), jnp.float32)
        return ()
    lax.fori_loop(0, SEG // 16, rzero, ())

    def radd(j, _):
        pltpu.sync_copy(stage_s.at[j, pl.ds(sid * SEG, SEG)], seg_v)

        def vadd(k, _):
            acc_v[pl.ds(k * 16, 16)] = acc_v[pl.ds(k * 16, 16)] + seg_v[pl.ds(k * 16, 16)]
            return ()
        lax.fori_loop(0, SEG // 16, vadd, ())
        return ()
    lax.fori_loop(0, NS, radd, ())

    pltpu.sync_copy(acc_v, deg_hbm.at[cid, pl.ds(sid * SEG, SEG)])


@functools.partial(
    pl.kernel,
    mesh=_mesh,
    out_type=jax.ShapeDtypeStruct((N, D), jnp.float32),
    scratch_types=[
        pltpu.VMEM((NCHUNK, CH), jnp.int32),
        pltpu.VMEM((NCHUNK, CH), jnp.int32),
        pltpu.VMEM((NB, CH, D2), jnp.float32),
        pltpu.VMEM((ROWS_T + 16,), jnp.float32),
        pltpu.VMEM((D2,), jnp.float32),
        pltpu.VMEM_SHARED((N, D2), jnp.float32),
        pltpu.SemaphoreType.DMA((NB,)),
        pltpu.SemaphoreType.DMA((NB,)),
    ],
    compiler_params=_sc_params,
)
def _msg_kernel(g2_hbm, src_hbm, dst_hbm, dinv_hbm, b_hbm, out_hbm,
                src_v, dst_v, rows_v, dinv_v, b_v, acc_s, gsem, ssem):
    cid = lax.axis_index("c")
    sid = lax.axis_index("s")

    pltpu.sync_copy(src_hbm.at[cid, sid], src_v)
    pltpu.sync_copy(dst_hbm.at[sid], dst_v)

    pltpu.sync_copy(dinv_hbm.at[sid], dinv_v.at[pl.ds(0, ROWS_T)])
    pltpu.sync_copy(b_hbm.at[cid], b_v)

    base = sid * ROWS_T
    pltpu.sync_copy(
        g2_hbm.at[pl.ds(cid * N + base, ROWS_T)], acc_s.at[pl.ds(base, ROWS_T)]
    )

    plsc.subcore_barrier()

    for b in range(NB):
        pltpu.async_copy(g2_hbm.at[src_v.at[b]], rows_v.at[b], gsem.at[b])

    def group_body(gi, _):
        for b in range(NB):
            c = gi * NB + b
            pltpu.make_async_copy(
                g2_hbm.at[src_v.at[c]], rows_v.at[b], gsem.at[b]
            ).wait()
            pltpu.async_copy(
                rows_v.at[b], acc_s.at[dst_v.at[c]], ssem.at[b], add=True
            )
        for b in range(NB):
            c = gi * NB + b
            pltpu.make_async_copy(
                rows_v.at[b], acc_s.at[dst_v.at[c]], ssem.at[b]
            ).wait()

            @pl.when(c + NB < NCHUNK)
            def _():
                pltpu.async_copy(
                    g2_hbm.at[src_v.at[c + NB]], rows_v.at[b], gsem.at[b]
                )
        return ()
    lax.fori_loop(0, NCHUNK // NB, group_body, ())

    for t in range(NTAIL):
        c = (NCHUNK // NB) * NB + t
        pltpu.make_async_copy(
            g2_hbm.at[src_v.at[c]], rows_v.at[t], gsem.at[t]
        ).wait()
        pltpu.async_copy(
            rows_v.at[t], acc_s.at[dst_v.at[c]], ssem.at[t], add=True
        )
    for t in range(NTAIL):
        c = (NCHUNK // NB) * NB + t
        pltpu.make_async_copy(
            rows_v.at[t], acc_s.at[dst_v.at[c]], ssem.at[t]
        ).wait()

    plsc.subcore_barrier()

    ebuf = rows_v.at[0]
    ecuts = [(0, CH), (CH, CH), (2 * CH, CH), (3 * CH, CH), (4 * CH, ROWS_T - 4 * CH)]
    for off, nr in ecuts:
        pltpu.sync_copy(acc_s.at[pl.ds(base + off, nr)], ebuf.at[pl.ds(0, nr)])

        def row_body(r, _):
            dv = dinv_v[pl.ds(off + r, 16)][0]
            for q in range(D2 // 16):
                v = ebuf[r, pl.ds(q * 16, 16)]
                s = v * dv + b_v[pl.ds(q * 16, 16)]
                ebuf[r, pl.ds(q * 16, 16)] = jnp.where(s >= 0.0, s, 0.2 * s)
            return ()
        lax.fori_loop(0, nr, row_body, ())

        pltpu.sync_copy(
            ebuf.at[pl.ds(0, nr)],
            out_hbm.at[pl.ds(base + off, nr), pl.ds(cid * D2, D2)],
        )


def _matmul_body(x_ref, w_ref, wc_ref, h2_ref):
    x0 = jnp.dot(x_ref[...], w_ref[...], preferred_element_type=jnp.float32)
    h2_ref[...] = jnp.dot(x0, wc_ref[0], preferred_element_type=jnp.float32)[None]


def _scale_body(h2_ref, deg0_ref, deg1_ref, g2_ref, dinv_ref):
    dinv = lax.rsqrt(deg0_ref[...] + deg1_ref[...] + 1.0)
    g2_ref[...] = h2_ref[...] * dinv[None]
    dinv_ref[...] = dinv


_BLK = 1000


def kernel(x, edge_index, W, Wc, b):
    src = edge_index[0]
    dst_flat = edge_index[1]
    src_both = jnp.stack([src, src + N]).reshape(NC, NS, NCHUNK, CH)
    dst = dst_flat.reshape(NS, NCHUNK, CH)

    deg2 = _deg_kernel(dst_flat)
    deg0 = deg2[0].reshape(NPAD, 1)
    deg1 = deg2[1].reshape(NPAD, 1)

    h2 = pl.pallas_call(
        _matmul_body,
        grid=(NC, N // _BLK),
        in_specs=[
            pl.BlockSpec((_BLK, D), lambda j, i: (i, 0)),
            pl.BlockSpec((D, D), lambda j, i: (0, 0)),
            pl.BlockSpec((1, D, D2), lambda j, i: (j, 0, 0)),
        ],
        out_specs=pl.BlockSpec((1, _BLK, D2), lambda j, i: (j, i, 0)),
        out_shape=jax.ShapeDtypeStruct((NC, N, D2), jnp.float32),
    )(x, W, Wc.reshape(D, NC, D2).transpose(1, 0, 2))

    g2, dinv = pl.pallas_call(
        _scale_body,
        grid=(NC, N // _BLK),
        in_specs=[
            pl.BlockSpec((1, _BLK, D2), lambda j, i: (j, i, 0)),
            pl.BlockSpec((_BLK, 1), lambda j, i: (i, 0)),
            pl.BlockSpec((_BLK, 1), lambda j, i: (i, 0)),
        ],
        out_specs=(
            pl.BlockSpec((1, _BLK, D2), lambda j, i: (j, i, 0)),
            pl.BlockSpec((_BLK, 1), lambda j, i: (i, 0)),
        ),
        out_shape=(
            jax.ShapeDtypeStruct((NC, N, D2), jnp.float32),
            jax.ShapeDtypeStruct((N, 1), jnp.float32),
        ),
    )(h2, deg0, deg1)

    out = _msg_kernel(
        g2.reshape(NC * N, D2),
        src_both,
        dst,
        dinv.reshape(NS, ROWS_T),
        b.reshape(NC, D2),
    )
    return out


# speedup vs baseline: 1.7111x; 1.1115x over previous
import functools

import jax
import jax.numpy as jnp
from jax import lax
from jax.experimental import pallas as pl
from jax.experimental.pallas import tpu as pltpu
from jax.experimental.pallas import tpu_sc as plsc

N = 10000
E = 320000
D = 128

NC = 2
NS = 16
NW = NC * NS
EW = E // NW
CH = 80
NCHUNK = EW // CH
NB = 3
NTAIL = NCHUNK - (NCHUNK // NB) * NB

NPAD = 10240
SEG = NPAD // NS

ROWS_T = N // NS

E_T = E // (NC * NS)
K1_CH = 2000
K1_NCHUNK = E_T // K1_CH

_mesh = plsc.VectorSubcoreMesh(core_axis_name="c", subcore_axis_name="s")
_sc_params = pltpu.CompilerParams(
    needs_layout_passes=False, use_tc_tiling_on_sc=False
)


@functools.partial(
    pl.kernel,
    mesh=_mesh,
    out_type=jax.ShapeDtypeStruct((NC, NPAD), jnp.float32),
    scratch_types=[
        pltpu.VMEM((K1_CH,), jnp.int32),
        pltpu.VMEM((NPAD,), jnp.float32),
        pltpu.VMEM((SEG,), jnp.float32),
        pltpu.VMEM((SEG,), jnp.float32),
        pltpu.VMEM_SHARED((NS, NPAD), jnp.float32),
    ],
    compiler_params=_sc_params,
)
def _deg_kernel(dst_hbm, deg_hbm, idx_v, hist_v, seg_v, acc_v, stage_s):
    cid = lax.axis_index("c")
    sid = lax.axis_index("s")
    ones = jnp.ones((16,), jnp.float32)

    def zero_body(i, _):
        hist_v[pl.ds(i * 16, 16)] = jnp.zeros((16,), jnp.float32)
        return ()
    lax.fori_loop(0, NPAD // 16, zero_body, ())

    def chunk_body(j, _):
        base = (cid * NS + sid) * E_T + j * K1_CH
        pltpu.sync_copy(dst_hbm.at[pl.ds(base, K1_CH)], idx_v)

        def vec_body(k, _):
            idx = idx_v[pl.ds(k * 16, 16)]
            plsc.addupdate_scatter(hist_v, [idx], ones)
            return ()
        lax.fori_loop(0, K1_CH // 16, vec_body, ())
        return ()
    lax.fori_loop(0, K1_NCHUNK, chunk_body, ())

    pltpu.sync_copy(hist_v, stage_s.at[sid])

    plsc.subcore_barrier()

    def rzero(i, _):
        acc_v[pl.ds(i * 16, 16)] = jnp.zeros((16,), jnp.float32)
        return ()
    lax.fori_loop(0, SEG // 16, rzero, ())

    def radd(j, _):
        pltpu.sync_copy(stage_s.at[j, pl.ds(sid * SEG, SEG)], seg_v)

        def vadd(k, _):
            acc_v[pl.ds(k * 16, 16)] = acc_v[pl.ds(k * 16, 16)] + seg_v[pl.ds(k * 16, 16)]
            return ()
        lax.fori_loop(0, SEG // 16, vadd, ())
        return ()
    lax.fori_loop(0, NS, radd, ())

    pltpu.sync_copy(acc_v, deg_hbm.at[cid, pl.ds(sid * SEG, SEG)])


@functools.partial(
    pl.kernel,
    mesh=_mesh,
    out_type=(
        jax.ShapeDtypeStruct((N, D), jnp.float32),
        jax.ShapeDtypeStruct((N, D), jnp.float32),
    ),
    scratch_types=[
        pltpu.VMEM((NCHUNK, CH), jnp.int32),
        pltpu.VMEM((NCHUNK, CH), jnp.int32),
        pltpu.VMEM((NB, CH, D), jnp.float32),
        pltpu.VMEM_SHARED((N, D), jnp.float32),
        pltpu.SemaphoreType.DMA((NB,)),
        pltpu.SemaphoreType.DMA((NB,)),
    ],
    compiler_params=_sc_params,
)
def _msg_kernel(g_hbm, src_hbm, dst_hbm, zeros_hbm, p0_hbm, p1_hbm,
                src_v, dst_v, rows_v, acc_s, gsem, ssem):
    cid = lax.axis_index("c")
    sid = lax.axis_index("s")
    w = sid * NC + cid

    pltpu.sync_copy(src_hbm.at[w], src_v)
    pltpu.sync_copy(dst_hbm.at[w], dst_v)

    base = sid * ROWS_T

    @pl.when(cid == 0)
    def _():
        pltpu.sync_copy(g_hbm.at[pl.ds(base, ROWS_T)], acc_s.at[pl.ds(base, ROWS_T)])

    @pl.when(cid != 0)
    def _():
        pltpu.sync_copy(zeros_hbm.at[pl.ds(base, ROWS_T)], acc_s.at[pl.ds(base, ROWS_T)])

    plsc.subcore_barrier()

    for b in range(NB):
        pltpu.async_copy(g_hbm.at[src_v.at[b]], rows_v.at[b], gsem.at[b])

    def group_body(gi, _):
        for b in range(NB):
            c = gi * NB + b
            pltpu.make_async_copy(
                g_hbm.at[src_v.at[c]], rows_v.at[b], gsem.at[b]
            ).wait()
            pltpu.async_copy(
                rows_v.at[b], acc_s.at[dst_v.at[c]], ssem.at[b], add=True
            )
        for b in range(NB):
            c = gi * NB + b
            pltpu.make_async_copy(
                rows_v.at[b], acc_s.at[dst_v.at[c]], ssem.at[b]
            ).wait()

            @pl.when(c + NB < NCHUNK)
            def _():
                pltpu.async_copy(
                    g_hbm.at[src_v.at[c + NB]], rows_v.at[b], gsem.at[b]
                )
        return ()
    lax.fori_loop(0, NCHUNK // NB, group_body, ())

    for t in range(NTAIL):
        c = (NCHUNK // NB) * NB + t
        pltpu.make_async_copy(
            g_hbm.at[src_v.at[c]], rows_v.at[t], gsem.at[t]
        ).wait()
        pltpu.async_copy(
            rows_v.at[t], acc_s.at[dst_v.at[c]], ssem.at[t], add=True
        )
    for t in range(NTAIL):
        c = (NCHUNK // NB) * NB + t
        pltpu.make_async_copy(
            rows_v.at[t], acc_s.at[dst_v.at[c]], ssem.at[t]
        ).wait()

    plsc.subcore_barrier()

    @pl.when(cid == 0)
    def _():
        pltpu.sync_copy(acc_s.at[pl.ds(base, ROWS_T)], p0_hbm.at[pl.ds(base, ROWS_T)])

    @pl.when(cid != 0)
    def _():
        pltpu.sync_copy(acc_s.at[pl.ds(base, ROWS_T)], p1_hbm.at[pl.ds(base, ROWS_T)])


def _transform_body(x_ref, w_ref, wc_ref, deg0_ref, deg1_ref, g_ref, dinv_ref):
    x0 = jnp.dot(x_ref[...], w_ref[...], preferred_element_type=jnp.float32)
    h = jnp.dot(x0, wc_ref[...], preferred_element_type=jnp.float32)
    dinv = lax.rsqrt(deg0_ref[...] + deg1_ref[...] + 1.0)
    g_ref[...] = h * dinv
    dinv_ref[...] = dinv


def _epilogue_body(p0_ref, p1_ref, dinv_ref, b_ref, out_ref):
    s = (p0_ref[...] + p1_ref[...]) * dinv_ref[...] + b_ref[...]
    out_ref[...] = jnp.where(s >= 0, s, 0.2 * s)


_BLK = 1000


def kernel(x, edge_index, W, Wc, b):
    src = edge_index[0].reshape(NW, NCHUNK, CH)
    dst_flat = edge_index[1]
    dst = dst_flat.reshape(NW, NCHUNK, CH)

    deg2 = _deg_kernel(dst_flat)
    deg0 = deg2[0].reshape(NPAD, 1)
    deg1 = deg2[1].reshape(NPAD, 1)

    g, dinv = pl.pallas_call(
        _transform_body,
        grid=(N // _BLK,),
        in_specs=[
            pl.BlockSpec((_BLK, D), lambda i: (i, 0)),
            pl.BlockSpec((D, D), lambda i: (0, 0)),
            pl.BlockSpec((D, D), lambda i: (0, 0)),
            pl.BlockSpec((_BLK, 1), lambda i: (i, 0)),
            pl.BlockSpec((_BLK, 1), lambda i: (i, 0)),
        ],
        out_specs=(
            pl.BlockSpec((_BLK, D), lambda i: (i, 0)),
            pl.BlockSpec((_BLK, 1), lambda i: (i, 0)),
        ),
        out_shape=(
            jax.ShapeDtypeStruct((N, D), jnp.float32),
            jax.ShapeDtypeStruct((N, 1), jnp.float32),
        ),
    )(x, W, Wc, deg0, deg1)

    zeros = jnp.zeros((N, D), jnp.float32)
    p0, p1 = _msg_kernel(g, src, dst, zeros)

    out = pl.pallas_call(
        _epilogue_body,
        grid=(N // _BLK,),
        in_specs=[
            pl.BlockSpec((_BLK, D), lambda i: (i, 0)),
            pl.BlockSpec((_BLK, D), lambda i: (i, 0)),
            pl.BlockSpec((_BLK, 1), lambda i: (i, 0)),
            pl.BlockSpec((1, D), lambda i: (0, 0)),
        ],
        out_specs=pl.BlockSpec((_BLK, D), lambda i: (i, 0)),
        out_shape=jax.ShapeDtypeStruct((N, D), jnp.float32),
    )(p0, p1, dinv, b.reshape(1, D))

    return out


# speedup vs baseline: 1.8406x; 1.0757x over previous
import functools

import jax
import jax.numpy as jnp
from jax import lax
from jax.experimental import pallas as pl
from jax.experimental.pallas import tpu as pltpu
from jax.experimental.pallas import tpu_sc as plsc

N = 10000
E = 320000
D = 128

NC = 2
NS = 16
NW = NC * NS
EW = E // NW
CH = 40
NCHUNK = EW // CH
NB = 6
NTAIL = NCHUNK - (NCHUNK // NB) * NB

NPAD = 10240
SEG = NPAD // NS

ROWS_T = N // NS

E_T = E // (NC * NS)

_mesh = plsc.VectorSubcoreMesh(core_axis_name="c", subcore_axis_name="s")
_sc_params = pltpu.CompilerParams(
    needs_layout_passes=False, use_tc_tiling_on_sc=False
)


@functools.partial(
    pl.kernel,
    mesh=_mesh,
    out_type=jax.ShapeDtypeStruct((NC, NPAD), jnp.float32),
    scratch_types=[
        pltpu.VMEM((E_T,), jnp.int32),
        pltpu.VMEM((NPAD,), jnp.float32),
        pltpu.VMEM((NS, SEG), jnp.float32),
        pltpu.VMEM((SEG,), jnp.float32),
        pltpu.VMEM_SHARED((NS, NPAD), jnp.float32),
    ],
    compiler_params=_sc_params,
)
def _deg_kernel(dst_hbm, deg_hbm, idx_v, hist_v, seg2_v, acc_v, stage_s):
    cid = lax.axis_index("c")
    sid = lax.axis_index("s")
    ones = jnp.ones((16,), jnp.float32)

    def zero_body(i, _):
        hist_v[pl.ds(i * 16, 16)] = jnp.zeros((16,), jnp.float32)
        return ()
    lax.fori_loop(0, NPAD // 16, zero_body, ())

    pltpu.sync_copy(dst_hbm.at[pl.ds((cid * NS + sid) * E_T, E_T)], idx_v)

    def vec_body(k, _):
        idx = idx_v[pl.ds(k * 16, 16)]
        plsc.addupdate_scatter(hist_v, [idx], ones)
        return ()
    lax.fori_loop(0, E_T // 16, vec_body, ())

    pltpu.sync_copy(hist_v, stage_s.at[sid])

    plsc.subcore_barrier()

    def rzero(i, _):
        acc_v[pl.ds(i * 16, 16)] = jnp.zeros((16,), jnp.float32)
        return ()
    lax.fori_loop(0, SEG // 16, rzero, ())

    pltpu.sync_copy(stage_s.at[:, pl.ds(sid * SEG, SEG)], seg2_v)

    def radd(j, _):
        def vadd(k, _):
            acc_v[pl.ds(k * 16, 16)] = acc_v[pl.ds(k * 16, 16)] + seg2_v[j, pl.ds(k * 16, 16)]
            return ()
        lax.fori_loop(0, SEG // 16, vadd, ())
        return ()
    lax.fori_loop(0, NS, radd, ())

    pltpu.sync_copy(acc_v, deg_hbm.at[cid, pl.ds(sid * SEG, SEG)])


@functools.partial(
    pl.kernel,
    mesh=_mesh,
    out_type=(
        jax.ShapeDtypeStruct((N, D), jnp.float32),
        jax.ShapeDtypeStruct((N, D), jnp.float32),
    ),
    scratch_types=[
        pltpu.VMEM((NCHUNK, CH), jnp.int32),
        pltpu.VMEM((NCHUNK, CH), jnp.int32),
        pltpu.VMEM((NB, CH, D), jnp.float32),
        pltpu.VMEM_SHARED((N, D), jnp.float32),
        pltpu.SemaphoreType.DMA((NB,)),
        pltpu.SemaphoreType.DMA((NB,)),
    ],
    compiler_params=_sc_params,
)
def _msg_kernel(g_hbm, src_hbm, dst_hbm, zeros_hbm, p0_hbm, p1_hbm,
                src_v, dst_v, rows_v, acc_s, gsem, ssem):
    cid = lax.axis_index("c")
    sid = lax.axis_index("s")
    w = sid * NC + cid

    pltpu.sync_copy(src_hbm.at[w], src_v)
    pltpu.sync_copy(dst_hbm.at[w], dst_v)

    base = sid * ROWS_T

    @pl.when(cid == 0)
    def _():
        pltpu.sync_copy(g_hbm.at[pl.ds(base, ROWS_T)], acc_s.at[pl.ds(base, ROWS_T)])

    @pl.when(cid != 0)
    def _():
        pltpu.sync_copy(zeros_hbm, acc_s.at[pl.ds(base, ROWS_T)])

    plsc.subcore_barrier()

    for b in range(NB):
        pltpu.async_copy(g_hbm.at[src_v.at[b]], rows_v.at[b], gsem.at[b])

    def group_body(gi, _):
        for b in range(NB):
            c = gi * NB + b
            pltpu.make_async_copy(
                g_hbm.at[src_v.at[c]], rows_v.at[b], gsem.at[b]
            ).wait()
            pltpu.async_copy(
                rows_v.at[b], acc_s.at[dst_v.at[c]], ssem.at[b], add=True
            )
        for b in range(NB):
            c = gi * NB + b
            pltpu.make_async_copy(
                rows_v.at[b], acc_s.at[dst_v.at[c]], ssem.at[b]
            ).wait()

            @pl.when(c + NB < NCHUNK)
            def _():
                pltpu.async_copy(
                    g_hbm.at[src_v.at[c + NB]], rows_v.at[b], gsem.at[b]
                )
        return ()
    lax.fori_loop(0, NCHUNK // NB, group_body, ())

    for t in range(NTAIL):
        c = (NCHUNK // NB) * NB + t
        pltpu.make_async_copy(
            g_hbm.at[src_v.at[c]], rows_v.at[t], gsem.at[t]
        ).wait()
        pltpu.async_copy(
            rows_v.at[t], acc_s.at[dst_v.at[c]], ssem.at[t], add=True
        )
    for t in range(NTAIL):
        c = (NCHUNK // NB) * NB + t
        pltpu.make_async_copy(
            rows_v.at[t], acc_s.at[dst_v.at[c]], ssem.at[t]
        ).wait()

    plsc.subcore_barrier()

    @pl.when(cid == 0)
    def _():
        pltpu.sync_copy(acc_s.at[pl.ds(base, ROWS_T)], p0_hbm.at[pl.ds(base, ROWS_T)])

    @pl.when(cid != 0)
    def _():
        pltpu.sync_copy(acc_s.at[pl.ds(base, ROWS_T)], p1_hbm.at[pl.ds(base, ROWS_T)])


def _transform_body(x_ref, w_ref, wc_ref, deg0_ref, deg1_ref, g_ref, dinv_ref):
    x0 = jnp.dot(x_ref[...], w_ref[...], preferred_element_type=jnp.float32)
    h = jnp.dot(x0, wc_ref[...], preferred_element_type=jnp.float32)
    dinv = lax.rsqrt(deg0_ref[...] + deg1_ref[...] + 1.0)
    g_ref[...] = h * dinv
    dinv_ref[...] = dinv


def _epilogue_body(p0_ref, p1_ref, dinv_ref, b_ref, out_ref):
    s = (p0_ref[...] + p1_ref[...]) * dinv_ref[...] + b_ref[...]
    out_ref[...] = jnp.where(s >= 0, s, 0.2 * s)


_BLK = 1000


def kernel(x, edge_index, W, Wc, b):
    src = edge_index[0].reshape(NW, NCHUNK, CH)
    dst_flat = edge_index[1]
    dst = dst_flat.reshape(NW, NCHUNK, CH)

    deg2 = _deg_kernel(dst_flat)
    deg0 = deg2[0].reshape(NPAD, 1)
    deg1 = deg2[1].reshape(NPAD, 1)

    g, dinv = pl.pallas_call(
        _transform_body,
        grid=(N // _BLK,),
        in_specs=[
            pl.BlockSpec((_BLK, D), lambda i: (i, 0)),
            pl.BlockSpec((D, D), lambda i: (0, 0)),
            pl.BlockSpec((D, D), lambda i: (0, 0)),
            pl.BlockSpec((_BLK, 1), lambda i: (i, 0)),
            pl.BlockSpec((_BLK, 1), lambda i: (i, 0)),
        ],
        out_specs=(
            pl.BlockSpec((_BLK, D), lambda i: (i, 0)),
            pl.BlockSpec((_BLK, 1), lambda i: (i, 0)),
        ),
        out_shape=(
            jax.ShapeDtypeStruct((N, D), jnp.float32),
            jax.ShapeDtypeStruct((N, 1), jnp.float32),
        ),
    )(x, W, Wc, deg0, deg1)

    zeros = jnp.zeros((ROWS_T, D), jnp.float32)
    p0, p1 = _msg_kernel(g, src, dst, zeros)

    out = pl.pallas_call(
        _epilogue_body,
        grid=(N // _BLK,),
        in_specs=[
            pl.BlockSpec((_BLK, D), lambda i: (i, 0)),
            pl.BlockSpec((_BLK, D), lambda i: (i, 0)),
            pl.BlockSpec((_BLK, 1), lambda i: (i, 0)),
            pl.BlockSpec((1, D), lambda i: (0, 0)),
        ],
        out_specs=pl.BlockSpec((_BLK, D), lambda i: (i, 0)),
        out_shape=jax.ShapeDtypeStruct((N, D), jnp.float32),
    )(p0, p1, dinv, b.reshape(1, D))

    return out
